# baseline (speedup 1.0000x reference)
import jax
import jax.numpy as jnp
from jax import lax
from jax.experimental import pallas as pl
from jax.experimental.pallas import tpu as pltpu
from jax.experimental.pallas import tpu_sc as plsc

BATCH = 16384
FEAT = 129
NFEAT = FEAT - 1
EMBED_DIM = 64
OUT_DIM = NFEAT + EMBED_DIM
TPAD = 128
TROWS = 1024

NC = 2
NS = 16
L = 16
NW = NC * NS

ROWS_PER_W = BATCH // NW
CHUNK = 32
NCHUNK = ROWS_PER_W // CHUNK
NBUF = 6


def _sc_body(inputs_hbm, idx_hbm, table_hbm, out_hbm,
             idx_v, emb_v, out_v, tstage_v, spt, sem_f, sem_g, sem_o):
    sid = lax.axis_index("s")
    wid = sid * NC + lax.axis_index("c")
    base = wid * ROWS_PER_W

    def feat_issue(ch, b):
        rb = base + ch * CHUNK
        return pltpu.async_copy(
            inputs_hbm.at[pl.ds(rb, CHUNK), pl.ds(0, NFEAT)],
            out_v.at[b, :, pl.ds(0, NFEAT)], sem_f)

    def g_issue(ch, b):
        return pltpu.async_copy(
            spt.at[idx_v.at[pl.ds(ch * CHUNK, CHUNK)]],
            emb_v.at[b], sem_g)

    feat_ds = [feat_issue(c, c) for c in range(NBUF - 1)] + [None]
    trows = TROWS // NS
    pltpu.sync_copy(table_hbm.at[pl.ds(sid * trows, trows), :], tstage_v)
    pltpu.sync_copy(tstage_v, spt.at[pl.ds(sid * trows, trows), :])
    pltpu.sync_copy(idx_hbm.at[pl.ds(base, ROWS_PER_W)], idx_v)
    plsc.subcore_barrier()
    g_ds = [g_issue(c, c) for c in range(NBUF - 1)] + [None]
    out_ds = [None] * NBUF
    for ch in range(NCHUNK):
        b = ch % NBUF
        g_ds[b].wait()
        feat_ds[b].wait()

        @plsc.parallel_loop(0, CHUNK, 1, unroll=4)
        def fold(r):
            for c in range(EMBED_DIM // L):
                out_v[b, r, pl.ds(NFEAT + c * L, L)] = \
                    emb_v[b, r, pl.ds(c * L, L)]
        out_ds[b] = pltpu.async_copy(
            out_v.at[b], out_hbm.at[pl.ds(base + ch * CHUNK, CHUNK), :],
            sem_o)
        nch = ch + (NBUF - 1)
        if nch < NCHUNK:
            nb = nch % NBUF
            if out_ds[nb] is not None:
                out_ds[nb].wait()
            feat_ds[nb] = feat_issue(nch, nb)
            g_ds[nb] = g_issue(nch, nb)
    for b in range(NBUF):
        if out_ds[b] is not None:
            out_ds[b].wait()


@jax.jit
def _personalized_input(inputs, table):
    mesh = plsc.VectorSubcoreMesh(
        core_axis_name="c", subcore_axis_name="s",
        num_cores=NC, num_subcores=NS)
    call = pl.kernel(
        _sc_body,
        out_type=jax.ShapeDtypeStruct((BATCH, OUT_DIM), jnp.float32),
        mesh=mesh,
        compiler_params=pltpu.CompilerParams(use_tc_tiling_on_sc=True),
        scratch_types=[
            pltpu.VMEM((ROWS_PER_W,), jnp.int32),
            pltpu.VMEM((NBUF, CHUNK, TPAD), jnp.float32),
            pltpu.VMEM((NBUF, CHUNK, OUT_DIM), jnp.float32),
            pltpu.VMEM((TROWS // NS, TPAD), jnp.float32),
            pltpu.VMEM_SHARED((TROWS, TPAD), jnp.float32),
            pltpu.SemaphoreType.DMA,
            pltpu.SemaphoreType.DMA,
            pltpu.SemaphoreType.DMA,
        ],
    )
    table_pad = jnp.pad(
        table, ((0, TROWS - table.shape[0]), (0, TPAD - EMBED_DIM)))
    return call(inputs, inputs[:, -1].astype(jnp.int32), table_pad)


def kernel(inputs, table):
    return _personalized_input(inputs, table)


# speedup vs baseline: 1.0264x; 1.0264x over previous
import jax
import jax.numpy as jnp
from jax import lax
from jax.experimental import pallas as pl
from jax.experimental.pallas import tpu as pltpu
from jax.experimental.pallas import tpu_sc as plsc

BATCH = 16384
FEAT = 129
NFEAT = FEAT - 1
EMBED_DIM = 64
OUT_DIM = NFEAT + EMBED_DIM
TPAD = 128
TROWS = 1024

NC = 2
NS = 16
L = 16
NW = NC * NS

ROWS_PER_W = BATCH // NW
CHUNK = 64
NCHUNK = ROWS_PER_W // CHUNK
NBUF = 4


def _sc_body(inputs_hbm, idx_hbm, table_hbm, out_hbm,
             idx_v, emb_v, out_v, tstage_v, spt, sem_f, sem_g, sem_o):
    sid = lax.axis_index("s")
    wid = sid * NC + lax.axis_index("c")
    base = wid * ROWS_PER_W

    def feat_issue(ch, b):
        rb = base + ch * CHUNK
        return pltpu.async_copy(
            inputs_hbm.at[pl.ds(rb, CHUNK), pl.ds(0, NFEAT)],
            out_v.at[b, :, pl.ds(0, NFEAT)], sem_f)

    def g_issue(ch, b):
        return pltpu.async_copy(
            spt.at[idx_v.at[pl.ds(ch * CHUNK, CHUNK)]],
            emb_v.at[b], sem_g)

    feat_ds = [feat_issue(0, 0), feat_issue(1, 1), feat_issue(2, 2), None]
    idx_d = pltpu.async_copy(idx_hbm.at[pl.ds(base, ROWS_PER_W)], idx_v,
                             sem_o)
    trows = TROWS // NS
    half = trows // 2
    h0 = pltpu.async_copy(
        table_hbm.at[pl.ds(sid * trows, half), :],
        tstage_v.at[pl.ds(0, half), :], sem_f)
    h1 = pltpu.async_copy(
        table_hbm.at[pl.ds(sid * trows + half, half), :],
        tstage_v.at[pl.ds(half, half), :], sem_f)
    h0.wait()
    s0 = pltpu.async_copy(
        tstage_v.at[pl.ds(0, half), :],
        spt.at[pl.ds(sid * trows, half), :], sem_g)
    h1.wait()
    s1 = pltpu.async_copy(
        tstage_v.at[pl.ds(half, half), :],
        spt.at[pl.ds(sid * trows + half, half), :], sem_g)
    s0.wait()
    s1.wait()
    idx_d.wait()
    plsc.subcore_barrier()
    g_ds = [g_issue(0, 0), g_issue(1, 1), g_issue(2, 2), None]
    out_ds = [None, None, None, None]
    for ch in range(NCHUNK):
        b = ch % NBUF
        g_ds[b].wait()
        feat_ds[b].wait()

        @plsc.parallel_loop(0, CHUNK, 1, unroll=4)
        def fold(r):
            for c in range(EMBED_DIM // L):
                out_v[b, r, pl.ds(NFEAT + c * L, L)] = \
                    emb_v[b, r, pl.ds(c * L, L)]
        out_ds[b] = pltpu.async_copy(
            out_v.at[b], out_hbm.at[pl.ds(base + ch * CHUNK, CHUNK), :],
            sem_o)
        nch = ch + 3
        if nch < NCHUNK:
            nb = nch % NBUF
            if out_ds[nb] is not None:
                out_ds[nb].wait()
            feat_ds[nb] = feat_issue(nch, nb)
            g_ds[nb] = g_issue(nch, nb)
    for b in range(NBUF):
        if out_ds[b] is not None:
            out_ds[b].wait()


@jax.jit
def _personalized_input(inputs, table):
    mesh = plsc.VectorSubcoreMesh(
        core_axis_name="c", subcore_axis_name="s",
        num_cores=NC, num_subcores=NS)
    call = pl.kernel(
        _sc_body,
        out_type=jax.ShapeDtypeStruct((BATCH, OUT_DIM), jnp.float32),
        mesh=mesh,
        compiler_params=pltpu.CompilerParams(use_tc_tiling_on_sc=True),
        scratch_types=[
            pltpu.VMEM((ROWS_PER_W,), jnp.int32),
            pltpu.VMEM((NBUF, CHUNK, TPAD), jnp.float32),
            pltpu.VMEM((NBUF, CHUNK, OUT_DIM), jnp.float32),
            pltpu.VMEM((TROWS // NS, TPAD), jnp.float32),
            pltpu.VMEM_SHARED((TROWS, TPAD), jnp.float32),
            pltpu.SemaphoreType.DMA,
            pltpu.SemaphoreType.DMA,
            pltpu.SemaphoreType.DMA,
        ],
    )
    table_pad = jnp.pad(
        table, ((0, TROWS - table.shape[0]), (0, TPAD - EMBED_DIM)))
    return call(inputs, inputs[:, -1].astype(jnp.int32), table_pad)


def kernel(inputs, table):
    return _personalized_input(inputs, table)
